# Initial kernel scaffold; baseline (speedup 1.0000x reference)
import jax
import jax.numpy as jnp
from jax.experimental import pallas as pl


def kernel(u, i, item_attrs, P, Q, bu, bi, mu, attr_emb, W_fusion, b_fusion):
    raise NotImplementedError("write your pallas kernel here")


# speedup vs baseline: 1.5485x; 1.5485x over previous
import numpy as np
import jax
import jax.numpy as jnp
from jax import lax
from jax.experimental import pallas as pl
from jax.experimental.pallas import tpu as pltpu
from jax.experimental.pallas import tpu_sc as plsc

B = 16384
K = 32
D = 16
H = 20
NC, NS = 2, 16
NW = NC * NS
BPW = B // NW
CPW = BPW // 128
APW = BPW * H // 128
G = 4
GC = APW // G
GR = GC * 128

_j_over_h = (np.arange(BPW * H) // H).astype(np.int32)
_SIDX_ALL = np.stack(
    [(w // NC) * BPW + _j_over_h for w in range(NW)]
).reshape(NW, APW, 128)


def _sc_body(u_ref, i_ref, ia_ref, sidx_ref, P_ref, Q_ref, bu_ref, bi_ref,
             ae_ref, uf_out, if_out, as_out, buv_out, biv_out,
             iu_v, ii_v, ia_v, sx_v, uf_v, if_v, bu_v, bi_v, rows_v, acc_s,
             sem, sem2):
    c = lax.axis_index("c")
    s = lax.axis_index("s")
    wid = s * NC + c
    pltpu.sync_copy(u_ref.at[wid], iu_v)
    pltpu.sync_copy(i_ref.at[wid], ii_v)
    pltpu.sync_copy(ia_ref.at[wid], ia_v)
    pltpu.sync_copy(sidx_ref.at[wid], sx_v)

    handles = []
    for cc in range(CPW):
        dst = pl.ds(cc * 128, 128)
        handles.append(pltpu.async_copy(P_ref.at[iu_v.at[cc]], uf_v.at[dst], sem))
        handles.append(pltpu.async_copy(Q_ref.at[ii_v.at[cc]], if_v.at[dst], sem))
        handles.append(pltpu.async_copy(bu_ref.at[iu_v.at[cc]], bu_v.at[dst], sem))
        handles.append(pltpu.async_copy(bi_ref.at[ii_v.at[cc]], bi_v.at[dst], sem))

    def zero_body(j, carry):
        rows_v[j] = jnp.zeros((D,), jnp.float32)
        return carry
    lax.fori_loop(0, BPW, zero_body, 0)
    pltpu.sync_copy(rows_v.at[pl.ds(0, BPW)], acc_s.at[pl.ds(s * BPW, BPW)])

    def group_body(g, carry):
        gh = []
        for j in range(GC):
            gh.append(pltpu.async_copy(
                ae_ref.at[ia_v.at[g * GC + j]],
                rows_v.at[pl.ds(j * 128, 128)], sem2))
        for hnd in gh:
            hnd.wait()
        for j in range(GC):
            pltpu.sync_copy(rows_v.at[pl.ds(j * 128, 128)],
                            acc_s.at[sx_v.at[g * GC + j]], add=True)
        return carry
    lax.fori_loop(0, G, group_body, 0)

    for hnd in handles:
        hnd.wait()

    out = pl.ds(wid * BPW, BPW)
    pltpu.sync_copy(uf_v, uf_out.at[out])
    pltpu.sync_copy(if_v, if_out.at[out])
    pltpu.sync_copy(bu_v, buv_out.at[out])
    pltpu.sync_copy(bi_v, biv_out.at[out])
    pltpu.sync_copy(acc_s.at[pl.ds(s * BPW, BPW)], as_out.at[out])


def _tc_body(uf_ref, if_ref, as_ref, buv_ref, biv_ref, wt_ref, bf_ref,
             mu_ref, out_ref):
    avg = as_ref[...] * (1.0 / H)
    attr = jnp.dot(avg, wt_ref[...], preferred_element_type=jnp.float32)
    itf = if_ref[...] + attr + bf_ref[...]
    pred = jnp.sum(uf_ref[...] * itf, axis=1)
    out_ref[...] = pred + buv_ref[...] + biv_ref[...] + mu_ref[0, 0]


def _make_sc_call():
    f32 = jnp.float32
    return pl.kernel(
        _sc_body,
        out_type=[
            jax.ShapeDtypeStruct((B, K), f32),
            jax.ShapeDtypeStruct((B, K), f32),
            jax.ShapeDtypeStruct((B, D), f32),
            jax.ShapeDtypeStruct((B,), f32),
            jax.ShapeDtypeStruct((B,), f32),
        ],
        mesh=plsc.VectorSubcoreMesh(core_axis_name="c", subcore_axis_name="s"),
        scratch_types=[
            pltpu.VMEM((CPW, 128), jnp.int32),
            pltpu.VMEM((CPW, 128), jnp.int32),
            pltpu.VMEM((APW, 128), jnp.int32),
            pltpu.VMEM((APW, 128), jnp.int32),
            pltpu.VMEM((BPW, K), f32),
            pltpu.VMEM((BPW, K), f32),
            pltpu.VMEM((BPW,), f32),
            pltpu.VMEM((BPW,), f32),
            pltpu.VMEM((GR, D), f32),
            pltpu.VMEM_SHARED((NS * BPW, D), f32),
            pltpu.SemaphoreType.DMA,
            pltpu.SemaphoreType.DMA,
        ],
        compiler_params=pltpu.CompilerParams(use_tc_tiling_on_sc=False),
    )


def kernel(u, i, item_attrs, P, Q, bu, bi, mu, attr_emb, W_fusion, b_fusion):
    u3 = u.astype(jnp.int32).reshape(NW, CPW, 128)
    i3 = i.astype(jnp.int32).reshape(NW, CPW, 128)
    ia3 = item_attrs.astype(jnp.int32).reshape(NW, APW, 128)
    sidx = jnp.asarray(_SIDX_ALL)

    f32 = jnp.float32
    uf, itf, asum, buv, biv = _make_sc_call()(
        u3, i3, ia3, sidx, P, Q, bu.reshape(-1), bi.reshape(-1), attr_emb)

    TB = 2048
    combine = pl.pallas_call(
        _tc_body,
        grid=(B // TB,),
        in_specs=[
            pl.BlockSpec((TB, K), lambda j: (j, 0)),
            pl.BlockSpec((TB, K), lambda j: (j, 0)),
            pl.BlockSpec((TB, D), lambda j: (j, 0)),
            pl.BlockSpec((TB,), lambda j: (j,)),
            pl.BlockSpec((TB,), lambda j: (j,)),
            pl.BlockSpec((D, K), lambda j: (0, 0)),
            pl.BlockSpec((1, K), lambda j: (0, 0)),
            pl.BlockSpec((1, 1), lambda j: (0, 0)),
        ],
        out_specs=pl.BlockSpec((TB,), lambda j: (j,)),
        out_shape=jax.ShapeDtypeStruct((B,), f32),
    )
    return combine(uf, itf, asum, buv, biv, W_fusion.T.astype(f32),
                   b_fusion.reshape(1, K), mu.reshape(1, 1))
